# baseline (speedup 1.0000x reference)
import functools

import numpy as np
import jax
import jax.numpy as jnp
from jax import lax
from jax.experimental import pallas as pl
from jax.experimental.pallas import tpu as pltpu
from jax.experimental.pallas import tpu_sc as plsc

B = 16
T = 1024
N = T - 1
D = T + 1
DP = 1040
R = 32
NCHUNK = 512 // R
W = 1072
NBLK = DP // 16
GQ = 5
GB = NBLK // GQ


def _inv_counts() -> np.ndarray:
    si = np.arange(DP)
    cnt = N - np.abs(si - (D // 2))
    return np.where(si < D, 1.0 / np.maximum(cnt, 1), 0.0).astype(np.float32)


_INVC = _inv_counts()


def _win_h0(ch):
    w = min(544 + 32 * ch, 1024)
    return 0, w, 1056 - w


def _win_h1(ch):
    w = 1024 - 32 * ch
    return 32 * ch, w, 0


_mesh = plsc.VectorSubcoreMesh(core_axis_name="c", subcore_axis_name="s")


@functools.partial(
    pl.kernel,
    out_type=jax.ShapeDtypeStruct((B, DP), jnp.float32),
    mesh=_mesh,
    scratch_types=[
        pltpu.VMEM((R, W), jnp.float32),
        pltpu.VMEM((R, W), jnp.float32),
        pltpu.VMEM((DP,), jnp.float32),
        pltpu.VMEM((DP,), jnp.float32),
        pltpu.VMEM((DP,), jnp.float32),
        pltpu.VMEM((DP,), jnp.float32),
        pltpu.VMEM_SHARED((16, DP), jnp.float32),
        pltpu.SemaphoreType.DMA,
        pltpu.SemaphoreType.DMA,
    ],
    compiler_params=pltpu.CompilerParams(
        use_tc_tiling_on_sc=False, needs_layout_passes=False),
)
def _diag_mean_sc(in_hbm, invc_hbm, out_hbm, rows0_ref, rows1_ref, acc_ref,
                  tmp_ref, invc_ref, obuf_ref, shared_ref, sem0, sem1):
    _ZERO16 = jnp.zeros((16,), jnp.float32)
    lastcol_mask = jnp.where(
        lax.iota(jnp.int32, 16) < 15, jnp.float32(1.0), jnp.float32(0.0))
    c = lax.axis_index("c")
    s = lax.axis_index("s")
    half = s % 2
    b = c * 8 + s // 2

    bufs = (rows0_ref, rows1_ref)
    sems = (sem0, sem1)

    def dma(ch):
        rbuf = bufs[ch % 2]
        sem = sems[ch % 2]
        xlo0, w0, p0 = _win_h0(ch)
        xlo1, w1, p1 = _win_h1(ch)

        class _Both:
            def start(self):
                @pl.when(half == 0)
                def _():
                    pltpu.make_async_copy(
                        in_hbm.at[b, pl.ds(32 * ch, R), pl.ds(xlo0, w0)],
                        rbuf.at[:, pl.ds(p0, w0)], sem).start()

                @pl.when(half == 1)
                def _():
                    pltpu.make_async_copy(
                        in_hbm.at[b, pl.ds(512 + 32 * ch, R), pl.ds(xlo1, w1)],
                        rbuf.at[:, pl.ds(p1, w1)], sem).start()

            def wait(self):
                @pl.when(half == 0)
                def _():
                    pltpu.make_async_copy(
                        in_hbm.at[b, pl.ds(32 * ch, R), pl.ds(xlo0, w0)],
                        rbuf.at[:, pl.ds(p0, w0)], sem).wait()

                @pl.when(half == 1)
                def _():
                    pltpu.make_async_copy(
                        in_hbm.at[b, pl.ds(512 + 32 * ch, R), pl.ds(xlo1, w1)],
                        rbuf.at[:, pl.ds(p1, w1)], sem).wait()

        return _Both()

    def zr(r, carry):
        for k in range(W // 16):
            rows0_ref[r, pl.ds(16 * k, 16)] = _ZERO16
            rows1_ref[r, pl.ds(16 * k, 16)] = _ZERO16
        return carry

    lax.fori_loop(0, R, zr, 0)

    dma(0).start()
    dma(1).start()

    pltpu.sync_copy(invc_hbm, invc_ref)
    for k in range(NBLK):
        acc_ref[pl.ds(16 * k, 16)] = _ZERO16

    def compute(rbuf):
        def g_body(g, carry, rbuf=rbuf):
            si0 = GB * 16 * g
            accs = tuple(acc_ref[pl.ds(si0 + 16 * j, 16)] for j in range(GB))

            def row_body(r, accs, si0=si0, rbuf=rbuf):
                base = si0 + r
                return tuple(
                    accs[j] + rbuf[r, pl.ds(base + 16 * j, 16)]
                    for j in range(GB)
                )

            accs = lax.fori_loop(0, R, row_body, accs, unroll=4)
            for j in range(GB):
                acc_ref[pl.ds(si0 + 16 * j, 16)] = accs[j]
            return carry

        lax.fori_loop(0, GQ, g_body, 0)

    for ch in range(NCHUNK):
        rbuf = bufs[ch % 2]
        dma(ch).wait()

        if ch == NCHUNK - 1:
            @pl.when(half == 0)
            def _(rbuf=rbuf):
                def mask0(r, carry):
                    rbuf[r, pl.ds(1040, 16)] = (
                        rbuf[r, pl.ds(1040, 16)] * lastcol_mask)
                    return carry
                lax.fori_loop(0, R, mask0, 0)

        strip = 1024 - 32 * ch

        @pl.when(half == 1)
        def _(rbuf=rbuf, ch=ch, strip=strip):
            def fix1(r, carry):
                blk = pl.ds(1008 - 32 * ch, 16)
                rbuf[r, blk] = rbuf[r, blk] * lastcol_mask
                if ch >= 2:
                    for k in range(4):
                        rbuf[r, pl.ds(strip + 16 * k, 16)] = _ZERO16
                return carry
            lax.fori_loop(0, R, fix1, 0)

        if ch == NCHUNK - 1:
            @pl.when(half == 1)
            def _(rbuf=rbuf):
                for k in range(34):
                    rbuf[R - 1, pl.ds(16 * k, 16)] = _ZERO16

        compute(rbuf)
        if ch + 2 < NCHUNK:
            dma(ch + 2).start()

    pltpu.sync_copy(acc_ref, shared_ref.at[s])
    plsc.subcore_barrier()

    @pl.when(s % 2 == 0)
    def _():
        pltpu.sync_copy(shared_ref.at[s + 1], tmp_ref)
        tvec = _ZERO16
        for k in range(NBLK):
            o = pl.ds(16 * k, 16)
            m = (acc_ref[o] + tmp_ref[o]) * invc_ref[o]
            obuf_ref[o] = m
            tvec = tvec + m
        mu = jnp.sum(tvec) * jnp.float32(1.0 / D)
        for k in range(NBLK):
            o = pl.ds(16 * k, 16)
            obuf_ref[o] = mu - obuf_ref[o]
        pltpu.sync_copy(obuf_ref, out_hbm.at[b])


@jax.jit
def kernel(inputs):
    invc = jnp.asarray(_INVC)
    out = _diag_mean_sc(inputs, invc)
    return out[:, :D]
